# baseline (speedup 1.0000x reference)
import functools

import jax
import jax.numpy as jnp
from jax import lax
from jax.experimental import pallas as pl
from jax.experimental.pallas import tpu as pltpu
from jax.experimental.pallas import tpu_sc as plsc

N_ATOMS = 100000
N_IN = 128
N_HIDDEN = 64
N_MOL = 1024

LANES = 16
NWORKERS = 32

HALF0 = 61440
BLK0 = 12288
HALF1 = 40960
BLK1 = 10240
SPARE = 8


def _mlp_body(x_ref, w1t_ref, b1_ref, w2_ref, b2_ref, y_ref, *, blk, base_row):
    i = pl.program_id(0)
    h = lax.dot_general(
        x_ref[...], w1t_ref[...], (((1,), (1,)), ((), ())),
        preferred_element_type=jnp.float32,
    )
    h = h + b1_ref[...]
    h = h * jax.nn.sigmoid(h)
    y = jnp.dot(h, w2_ref[...], preferred_element_type=jnp.float32) + b2_ref[...]
    rows_blk = blk // 128
    yw = y.reshape(rows_blk, 128)
    rows = (
        base_row + i * blk
        + lax.broadcasted_iota(jnp.int32, (rows_blk, 128), 0) * 128
        + lax.broadcasted_iota(jnp.int32, (rows_blk, 128), 1)
    )
    y_ref[...] = jnp.where(rows < N_ATOMS, yw, 0.0)


def _mlp_phase(x, W1t, b1r, W2, b2r, blk, grid, base_row):
    blk_off = base_row // blk
    rows_blk = blk // 128
    return pl.pallas_call(
        functools.partial(_mlp_body, blk=blk, base_row=base_row),
        grid=(grid,),
        in_specs=[
            pl.BlockSpec((blk, N_IN), lambda i, o=blk_off: (i + o, 0)),
            pl.BlockSpec((N_HIDDEN, N_IN), lambda i: (0, 0)),
            pl.BlockSpec((1, N_HIDDEN), lambda i: (0, 0)),
            pl.BlockSpec((N_HIDDEN, 1), lambda i: (0, 0)),
            pl.BlockSpec((1, 1), lambda i: (0, 0)),
        ],
        out_specs=pl.BlockSpec((rows_blk, 128), lambda i: (i, 0)),
        out_shape=jax.ShapeDtypeStruct(
            (grid * rows_blk + SPARE, 128), jnp.float32
        ),
    )(x, W1t, b1r, W2, b2r)


def _sc_segment_sum(y_wide, idx, base_atom, natoms_phase):
    chunk = natoms_phase // NWORKERS
    crows = chunk // 128
    window = ((crows + 7 + 7) // 8) * 8
    real = max(0, min(natoms_phase, N_ATOMS - base_atom))
    full = real // chunk
    part = real - full * chunk
    assert part % LANES == 0 and chunk % LANES == 0 and base_atom % 8 == 0

    mesh = plsc.VectorSubcoreMesh(core_axis_name="c", subcore_axis_name="s")

    @functools.partial(
        pl.kernel,
        mesh=mesh,
        out_type=jax.ShapeDtypeStruct((NWORKERS, N_MOL), jnp.float32),
        scratch_types=[
            pltpu.VMEM((window, 128), jnp.float32),
            pltpu.VMEM((chunk,), jnp.int32),
            pltpu.VMEM((N_MOL,), jnp.float32),
        ],
        compiler_params=pltpu.CompilerParams(needs_layout_passes=False),
    )
    def body(y_hbm, idx_hbm, out_hbm, y_v, idx_v, acc_v):
        wid = lax.axis_index("s") * 2 + lax.axis_index("c")
        row0 = wid * crows
        base8 = (row0 // 8) * 8
        delta = row0 - base8
        pltpu.sync_copy(y_hbm.at[pl.ds(base8, window)], y_v)

        zero_i = jnp.zeros((LANES,), jnp.int32)

        if full < NWORKERS:
            @pl.when(wid < full)
            def _():
                pltpu.sync_copy(
                    idx_hbm.at[pl.ds(base_atom + wid * chunk, chunk)], idx_v
                )

            if part > 0:
                @pl.when(wid == full)
                def _():
                    def zpad_body(k, _):
                        idx_v[pl.ds(part + k * LANES, LANES)] = zero_i
                        return 0

                    lax.fori_loop(0, (chunk - part) // LANES, zpad_body, 0)
                    pltpu.sync_copy(
                        idx_hbm.at[pl.ds(base_atom + full * chunk, part)],
                        idx_v.at[pl.ds(0, part)],
                    )

            @pl.when(wid > full)
            def _():
                def zall_body(k, _):
                    idx_v[pl.ds(k * LANES, LANES)] = zero_i
                    return 0

                lax.fori_loop(0, chunk // LANES, zall_body, 0)
        else:
            pltpu.sync_copy(
                idx_hbm.at[pl.ds(base_atom + wid * chunk, chunk)], idx_v
            )

        zero = jnp.zeros((LANES,), jnp.float32)

        def zero_body(k, _):
            acc_v[pl.ds(k * LANES, LANES)] = zero
            return 0

        lax.fori_loop(0, N_MOL // LANES, zero_body, 0)

        def row_body(r, _):
            for c in range(128 // LANES):
                idx = idx_v[pl.ds(r * 128 + c * LANES, LANES)]
                val = y_v[delta + r, pl.ds(c * LANES, LANES)]
                plsc.addupdate_scatter(acc_v, [idx], val)
            return 0

        lax.fori_loop(0, crows, row_body, 0)
        pltpu.sync_copy(acc_v, out_hbm.at[wid])

    return body(y_wide, idx)


def _combine_body(p0_ref, p1_ref, o_ref):
    o_ref[...] = jnp.sum(p0_ref[...], axis=0, keepdims=True) + jnp.sum(
        p1_ref[...], axis=0, keepdims=True
    )


def _combine(p0, p1):
    return pl.pallas_call(
        _combine_body,
        out_shape=jax.ShapeDtypeStruct((1, N_MOL), jnp.float32),
    )(p0, p1)


def kernel(scalar_representation, idx_m, W1, b1, W2, b2):
    W1t = W1.T
    b1r = b1.reshape(1, N_HIDDEN)
    b2r = b2.reshape(1, 1)
    idx = idx_m.astype(jnp.int32)
    x = scalar_representation

    y0 = _mlp_phase(x, W1t, b1r, W2, b2r, BLK0, HALF0 // BLK0, 0)
    p0 = _sc_segment_sum(y0, idx, 0, HALF0)
    y1 = _mlp_phase(x, W1t, b1r, W2, b2r, BLK1, HALF1 // BLK1, HALF0)
    p1 = _sc_segment_sum(y1, idx, HALF0, HALF1)
    out = _combine(p0, p1)
    return out.reshape(N_MOL)
